# baseline (speedup 1.0000x reference)
import functools

import jax
import jax.numpy as jnp
from jax.experimental import pallas as pl
from jax.experimental.pallas import tpu as pltpu


def _conv3x3_kernel(x_ref, w_ref, b_ref, o_ref, *, B, Cin, Cout, H, W, RC):
    HW = H * W
    CS = RC * W
    K12 = 3 * Cin
    col = jax.lax.broadcasted_iota(jnp.int32, (1, CS), 1) % W
    m_left = (col != 0).astype(jnp.bfloat16)
    m_right = (col != (W - 1)).astype(jnp.bfloat16)
    zrow = jnp.zeros((Cin, W), jnp.bfloat16)
    bias = b_ref[...]
    w_all = w_ref[...]
    for b in range(B):
        xpad = jnp.concatenate(
            [zrow, x_ref[b].astype(jnp.bfloat16).reshape(Cin, HW), zrow],
            axis=1)
        for c in range(H // RC):
            base = c * CS
            r12 = jnp.concatenate(
                [xpad[:, base:base + CS],
                 xpad[:, base + W:base + W + CS],
                 xpad[:, base + 2 * W:base + 2 * W + CS]],
                axis=0)
            rm = jnp.concatenate([r12[:, :1], r12[:, :CS - 1]],
                                 axis=1) * m_left
            rp = jnp.concatenate([r12[:, 1:], r12[:, CS - 1:]],
                                 axis=1) * m_right
            rows = jnp.concatenate([rm, r12, rp], axis=0)
            y = jnp.dot(w_all, rows, preferred_element_type=jnp.float32)
            y = jnp.maximum(y + bias, 0.0)
            o_ref[b, :, c * RC:(c + 1) * RC, :] = y.reshape(Cout, RC, W)


def _forward(x_nchw, weight_oihw, bias_o, *, batch_tile, row_chunk):
    N, Cin, H, W = x_nchw.shape
    Cout, _, KH, KW = weight_oihw.shape
    HW = H * W
    w_all = jnp.transpose(weight_oihw, (0, 3, 2, 1)).reshape(
        Cout, KW * KH * Cin).astype(jnp.bfloat16)
    b_col = bias_o.reshape(Cout, 1)
    B = batch_tile
    grid = (N // B,)
    cost = pl.CostEstimate(
        flops=2 * N * Cout * (KW * KH * Cin) * HW,
        transcendentals=0,
        bytes_accessed=(x_nchw.size * 4 + w_all.size * 2 + Cout * 4
                        + N * Cout * HW * 4),
    )
    out = pl.pallas_call(
        functools.partial(_conv3x3_kernel, B=B, Cin=Cin, Cout=Cout,
                          H=H, W=W, RC=row_chunk),
        out_shape=jax.ShapeDtypeStruct((N, Cout, H, W), jnp.float32),
        grid=grid,
        in_specs=[
            pl.BlockSpec((B, Cin, H, W), lambda n: (n, 0, 0, 0)),
            pl.BlockSpec((Cout, KW * KH * Cin), lambda n: (0, 0)),
            pl.BlockSpec((Cout, 1), lambda n: (0, 0)),
        ],
        out_specs=pl.BlockSpec((B, Cout, H, W), lambda n: (n, 0, 0, 0)),
        compiler_params=pltpu.CompilerParams(
            dimension_semantics=("parallel",)),
        cost_estimate=cost,
    )(x_nchw, w_all, b_col)
    return out


def kernel(x_nchw, weight_oihw, bias_o):
    return _forward(x_nchw, weight_oihw, bias_o, batch_tile=8, row_chunk=16)


# speedup vs baseline: 1.1494x; 1.1494x over previous
import functools

import jax
import jax.numpy as jnp
from jax.experimental import pallas as pl
from jax.experimental.pallas import tpu as pltpu


def _conv3x3_kernel(x_ref, w_ref, b_ref, o_ref, *, B, Cin, Cout, H, W, RC):
    HW = H * W
    CS = RC * W
    col = jax.lax.broadcasted_iota(jnp.int32, (Cout, CS), 1) % W
    m_left = (col != 0).astype(jnp.float32)
    m_right = (col != (W - 1)).astype(jnp.float32)
    zrow = jnp.zeros((Cin, W), jnp.bfloat16)
    bias = b_ref[...]
    w_all = w_ref[...]
    for b in range(B):
        xpad = jnp.concatenate(
            [zrow, x_ref[b].astype(jnp.bfloat16).reshape(Cin, HW), zrow],
            axis=1)
        for c in range(H // RC):
            base = c * CS
            rows = jnp.concatenate(
                [xpad[:, base:base + CS],
                 xpad[:, base + W:base + W + CS],
                 xpad[:, base + 2 * W:base + 2 * W + CS]],
                axis=0)
            t = jnp.dot(w_all, rows, preferred_element_type=jnp.float32)
            t0, t1, t2 = t[:Cout], t[Cout:2 * Cout], t[2 * Cout:]
            s0 = jnp.concatenate([t0[:, :1], t0[:, :CS - 1]], axis=1)
            s2 = jnp.concatenate([t2[:, 1:], t2[:, CS - 1:]], axis=1)
            y = jnp.maximum(t1 + m_left * s0 + m_right * s2 + bias, 0.0)
            o_ref[b, :, c * RC:(c + 1) * RC, :] = y.reshape(Cout, RC, W)


def _forward(x_nchw, weight_oihw, bias_o, *, batch_tile, row_chunk):
    N, Cin, H, W = x_nchw.shape
    Cout, _, KH, KW = weight_oihw.shape
    HW = H * W
    w_all = jnp.transpose(weight_oihw, (3, 0, 2, 1)).reshape(
        KW * Cout, KH * Cin).astype(jnp.bfloat16)
    b_col = bias_o.reshape(Cout, 1)
    B = batch_tile
    grid = (N // B,)
    cost = pl.CostEstimate(
        flops=2 * N * (KW * Cout) * (KH * Cin) * HW,
        transcendentals=0,
        bytes_accessed=(x_nchw.size * 4 + w_all.size * 2 + Cout * 4
                        + N * Cout * HW * 4),
    )
    out = pl.pallas_call(
        functools.partial(_conv3x3_kernel, B=B, Cin=Cin, Cout=Cout,
                          H=H, W=W, RC=row_chunk),
        out_shape=jax.ShapeDtypeStruct((N, Cout, H, W), jnp.float32),
        grid=grid,
        in_specs=[
            pl.BlockSpec((B, Cin, H, W), lambda n: (n, 0, 0, 0)),
            pl.BlockSpec((KW * Cout, KH * Cin), lambda n: (0, 0)),
            pl.BlockSpec((Cout, 1), lambda n: (0, 0)),
        ],
        out_specs=pl.BlockSpec((B, Cout, H, W), lambda n: (n, 0, 0, 0)),
        compiler_params=pltpu.CompilerParams(
            dimension_semantics=("parallel",)),
        cost_estimate=cost,
    )(x_nchw, w_all, b_col)
    return out


def kernel(x_nchw, weight_oihw, bias_o):
    return _forward(x_nchw, weight_oihw, bias_o, batch_tile=16, row_chunk=16)
